# baseline (speedup 1.0000x reference)
import functools

import jax
import jax.numpy as jnp
from jax import lax
from jax.experimental import pallas as pl
from jax.experimental.pallas import tpu as pltpu
from jax.experimental.pallas import tpu_sc as plsc

EMBED_DIM = 64
CHUNK = 128
NUM_WORKERS = 32


def _make_gather(n_rows: int):
    chunks = n_rows // CHUNK
    cpw = chunks // NUM_WORKERS
    mesh = plsc.VectorSubcoreMesh(core_axis_name="c", subcore_axis_name="s")

    @functools.partial(
        pl.kernel,
        mesh=mesh,
        compiler_params=pltpu.CompilerParams(use_tc_tiling_on_sc=False),
        out_type=jax.ShapeDtypeStruct((n_rows, EMBED_DIM), jnp.float32),
        scratch_types=[
            pltpu.VMEM((cpw, CHUNK), jnp.int32),
            pltpu.VMEM((CHUNK, EMBED_DIM), jnp.float32),
            pltpu.SemaphoreType.DMA,
        ],
    )
    def gather_kernel(table_hbm, idx_hbm, out_hbm, idx_v, rows_v, sem):
        wid = lax.axis_index("s") * 2 + lax.axis_index("c")
        c0 = wid * cpw
        pltpu.sync_copy(idx_hbm.at[pl.ds(c0, cpw)], idx_v)

        def step(j, carry):
            pltpu.async_copy(table_hbm.at[idx_v.at[j]], rows_v, sem).wait()
            pltpu.sync_copy(rows_v, out_hbm.at[pl.ds((c0 + j) * CHUNK, CHUNK)])
            return carry

        lax.fori_loop(0, cpw, step, 0)

    return gather_kernel


def kernel(x, table):
    b, h = x.shape
    n = b * h
    idx = x.reshape(n // CHUNK, CHUNK).astype(jnp.int32)
    out = _make_gather(n)(table, idx)
    return out.reshape(b, h, EMBED_DIM)


# speedup vs baseline: 1.1163x; 1.1163x over previous
import functools

import jax
import jax.numpy as jnp
from jax import lax
from jax.experimental import pallas as pl
from jax.experimental.pallas import tpu as pltpu
from jax.experimental.pallas import tpu_sc as plsc

EMBED_DIM = 64
CHUNK = 128
K = 2
NBUF = 4
NUM_WORKERS = 32


def _make_gather(n_rows: int):
    chunks = n_rows // CHUNK
    cpw = chunks // NUM_WORKERS
    ngroups = cpw // K
    niters = ngroups // NBUF
    grp = K * CHUNK
    mesh = plsc.VectorSubcoreMesh(core_axis_name="c", subcore_axis_name="s")

    @functools.partial(
        pl.kernel,
        mesh=mesh,
        compiler_params=pltpu.CompilerParams(use_tc_tiling_on_sc=False),
        out_type=jax.ShapeDtypeStruct((n_rows, EMBED_DIM), jnp.float32),
        scratch_types=[
            pltpu.VMEM((cpw, CHUNK), jnp.int32),
            pltpu.VMEM((NBUF, grp, EMBED_DIM), jnp.float32),
            pltpu.SemaphoreType.DMA((NBUF,)),
            pltpu.SemaphoreType.DMA((NBUF,)),
        ],
    )
    def gather_kernel(table_hbm, idx_hbm, out_hbm, idx_v, rows_v, gsem, osem):
        wid = lax.axis_index("s") * 2 + lax.axis_index("c")
        c0 = wid * cpw
        pltpu.sync_copy(idx_hbm.at[pl.ds(c0, cpw)], idx_v)

        def gather_desc(g, b, t):
            return pltpu.make_async_copy(
                table_hbm.at[idx_v.at[g * K + t]],
                rows_v.at[b].at[pl.ds(t * CHUNK, CHUNK)],
                gsem.at[b],
            )

        def out_desc(g, b):
            return pltpu.make_async_copy(
                rows_v.at[b],
                out_hbm.at[pl.ds((c0 + g * K) * CHUNK, grp)],
                osem.at[b],
            )

        def fire(g, b):
            for t in range(K):
                gather_desc(g, b, t).start()

        def drain_then_out(g, b):
            for t in range(K):
                gather_desc(g, b, t).wait()
            out_desc(g, b).start()

        for b in range(NBUF):
            fire(b, b)

        def it(i, carry):
            for b in range(NBUF):
                g = i * NBUF + b
                drain_then_out(g, b)
                out_desc(g, b).wait()
                fire(g + NBUF, b)
            return carry

        lax.fori_loop(0, niters - 1, it, 0)

        for b in range(NBUF):
            g = (niters - 1) * NBUF + b
            drain_then_out(g, b)
        for b in range(NBUF):
            out_desc((niters - 1) * NBUF + b, b).wait()

    return gather_kernel


def kernel(x, table):
    b, h = x.shape
    n = b * h
    idx = x.reshape(n // CHUNK, CHUNK).astype(jnp.int32)
    out = _make_gather(n)(table, idx)
    return out.reshape(b, h, EMBED_DIM)
